# baseline (device time: 14760 ns/iter reference)
import jax
import jax.numpy as jnp
from jax import lax
from jax.experimental import pallas as pl
from jax.experimental.pallas import tpu as pltpu

N_DEV = 8
N_TOK = 512
D_IN = 256
D_OUT = 512
CHUNK = N_TOK // N_DEV
CAPACITY = 25
EXPERTS_PER_DEV = 2


def kernel(x, router_W, route_idx, expert_W):
    del router_W

    my_pos = lax.axis_index("i")
    r = route_idx[:, 0]

    def gate_for(e):
        m = (r == e).astype(jnp.int32)
        rank_excl = jnp.cumsum(m) - m
        keep = (m > 0) & (rank_excl < CAPACITY)
        return keep.astype(jnp.float32)

    gates = jnp.stack(
        [gate_for(EXPERTS_PER_DEV * my_pos + s) for s in range(EXPERTS_PER_DEV)],
        axis=1,
    )

    def body(
        x_ref,
        gates_ref,
        w_ref,
        out_ref,
        partial_ref,
        recv_buf,
        send_sems,
        recv_sems,
    ):
        p = lax.axis_index("i")

        barrier_sem = pltpu.get_barrier_semaphore()
        for d in range(1, N_DEV):
            pl.semaphore_signal(
                barrier_sem,
                inc=1,
                device_id=((p + d) % N_DEV,),
                device_id_type=pl.DeviceIdType.MESH,
            )
        pl.semaphore_wait(barrier_sem, N_DEV - 1)

        xb = x_ref[:, :].astype(jnp.bfloat16)
        gb = gates_ref[:, :].astype(jnp.bfloat16)
        x0 = xb * gb[:, 0:1]
        x1 = xb * gb[:, 1:2]
        acc32 = jnp.dot(
            x0,
            w_ref[0].astype(jnp.bfloat16),
            preferred_element_type=jnp.float32,
        ) + jnp.dot(
            x1,
            w_ref[1].astype(jnp.bfloat16),
            preferred_element_type=jnp.float32,
        )
        partial_ref[:, :] = acc32.astype(jnp.bfloat16)

        rdmas = []
        for d in range(1, N_DEV):
            t = (p + d) % N_DEV
            rdma = pltpu.make_async_remote_copy(
                src_ref=partial_ref.at[pl.ds(t * CHUNK, CHUNK), :],
                dst_ref=recv_buf.at[d - 1],
                send_sem=send_sems.at[d - 1],
                recv_sem=recv_sems.at[d - 1],
                device_id=(t,),
                device_id_type=pl.DeviceIdType.MESH,
            )
            rdma.start()
            rdmas.append(rdma)

        acc = partial_ref[pl.ds(p * CHUNK, CHUNK), :].astype(jnp.float32)
        for d in range(1, N_DEV):
            rdmas[d - 1].wait_recv()
            acc = acc + recv_buf[d - 1].astype(jnp.float32)
        out_ref[:, :] = acc

        for rdma in rdmas:
            rdma.wait_send()

    return pl.pallas_call(
        body,
        out_shape=jax.ShapeDtypeStruct((CHUNK, D_OUT), jnp.float32),
        in_specs=[
            pl.BlockSpec(memory_space=pltpu.VMEM),
            pl.BlockSpec(memory_space=pltpu.VMEM),
            pl.BlockSpec(memory_space=pltpu.VMEM),
        ],
        out_specs=pl.BlockSpec(memory_space=pltpu.VMEM),
        scratch_shapes=[
            pltpu.VMEM((N_TOK, D_OUT), jnp.bfloat16),
            pltpu.VMEM((N_DEV - 1, CHUNK, D_OUT), jnp.bfloat16),
            pltpu.SemaphoreType.DMA((N_DEV - 1,)),
            pltpu.SemaphoreType.DMA((N_DEV - 1,)),
        ],
        compiler_params=pltpu.CompilerParams(collective_id=0),
    )(x, gates, expert_W)


# device time: 12986 ns/iter; 1.1366x vs baseline; 1.1366x over previous
import jax
import jax.numpy as jnp
from jax import lax
from jax.experimental import pallas as pl
from jax.experimental.pallas import tpu as pltpu

N_DEV = 8
N_TOK = 512
D_IN = 256
D_OUT = 512
CHUNK = N_TOK // N_DEV
CAPACITY = 25
EXPERTS_PER_DEV = 2


def kernel(x, router_W, route_idx, expert_W):
    del router_W

    def body(
        x_ref,
        route_ref,
        w_ref,
        out_ref,
        partial_ref,
        recv_buf,
        send_sems,
        recv_sems,
    ):
        p = lax.axis_index("i")

        barrier_sem = pltpu.get_barrier_semaphore()
        for d in range(1, N_DEV):
            pl.semaphore_signal(
                barrier_sem,
                inc=1,
                device_id=((p + d) % N_DEV,),
                device_id_type=pl.DeviceIdType.MESH,
            )

        col = lax.broadcasted_iota(jnp.int32, (N_TOK, EXPERTS_PER_DEV), 1)
        m = route_ref[:, :] == (EXPERTS_PER_DEV * p + col)
        row_i = lax.broadcasted_iota(jnp.int32, (N_TOK, N_TOK), 0)
        col_i = lax.broadcasted_iota(jnp.int32, (N_TOK, N_TOK), 1)
        tril = (col_i < row_i).astype(jnp.bfloat16)
        rank = jnp.dot(
            tril, m.astype(jnp.bfloat16), preferred_element_type=jnp.float32
        )
        gates = jnp.where(m & (rank < CAPACITY), 1.0, 0.0).astype(jnp.bfloat16)

        xb = x_ref[:, :].astype(jnp.bfloat16)
        x0 = xb * gates[:, 0:1]
        x1 = xb * gates[:, 1:2]
        acc32 = jnp.dot(
            x0,
            w_ref[0].astype(jnp.bfloat16),
            preferred_element_type=jnp.float32,
        ) + jnp.dot(
            x1,
            w_ref[1].astype(jnp.bfloat16),
            preferred_element_type=jnp.float32,
        )
        partial_ref[:, :] = acc32.astype(jnp.bfloat16)

        pl.semaphore_wait(barrier_sem, N_DEV - 1)

        rdmas = []
        for d in range(1, N_DEV):
            t = (p + d) % N_DEV
            rdma = pltpu.make_async_remote_copy(
                src_ref=partial_ref.at[pl.ds(t * CHUNK, CHUNK), :],
                dst_ref=recv_buf.at[d - 1],
                send_sem=send_sems.at[d - 1],
                recv_sem=recv_sems.at[d - 1],
                device_id=(t,),
                device_id_type=pl.DeviceIdType.MESH,
            )
            rdma.start()
            rdmas.append(rdma)

        acc = partial_ref[pl.ds(p * CHUNK, CHUNK), :].astype(jnp.float32)
        for d in range(1, N_DEV):
            rdmas[d - 1].wait_recv()
            acc = acc + recv_buf[d - 1].astype(jnp.float32)
        out_ref[:, :] = acc

        for rdma in rdmas:
            rdma.wait_send()

    return pl.pallas_call(
        body,
        out_shape=jax.ShapeDtypeStruct((CHUNK, D_OUT), jnp.float32),
        in_specs=[
            pl.BlockSpec(memory_space=pltpu.VMEM),
            pl.BlockSpec(memory_space=pltpu.VMEM),
            pl.BlockSpec(memory_space=pltpu.VMEM),
        ],
        out_specs=pl.BlockSpec(memory_space=pltpu.VMEM),
        scratch_shapes=[
            pltpu.VMEM((N_TOK, D_OUT), jnp.bfloat16),
            pltpu.VMEM((N_DEV - 1, CHUNK, D_OUT), jnp.bfloat16),
            pltpu.SemaphoreType.DMA((N_DEV - 1,)),
            pltpu.SemaphoreType.DMA((N_DEV - 1,)),
        ],
        compiler_params=pltpu.CompilerParams(collective_id=0),
    )(x, route_idx, expert_W)


# device time: 12794 ns/iter; 1.1537x vs baseline; 1.0150x over previous
import jax
import jax.numpy as jnp
from jax import lax
from jax.experimental import pallas as pl
from jax.experimental.pallas import tpu as pltpu

N_DEV = 8
N_TOK = 512
D_IN = 256
D_OUT = 512
CHUNK = N_TOK // N_DEV
CAPACITY = 25
EXPERTS_PER_DEV = 2


def kernel(x, router_W, route_idx, expert_W):
    del router_W

    def body(
        x_ref,
        route_ref,
        w_ref,
        out_ref,
        partial_ref,
        gates_ref,
        recv_buf,
        send_sems,
        recv_sems,
        credit_sems,
    ):
        p = lax.axis_index("i")

        barrier_sem = pltpu.get_barrier_semaphore()
        pl.semaphore_signal(barrier_sem, inc=N_DEV - 1)
        pl.semaphore_wait(barrier_sem, N_DEV - 1)

        for d in range(1, N_DEV):
            pl.semaphore_signal(
                credit_sems.at[N_DEV - 1 - d],
                inc=1,
                device_id=((p + d) % N_DEV,),
                device_id_type=pl.DeviceIdType.MESH,
            )

        col = lax.broadcasted_iota(jnp.int32, (N_TOK, EXPERTS_PER_DEV), 1)
        m = route_ref[:, :] == (EXPERTS_PER_DEV * p + col)
        row_i = lax.broadcasted_iota(jnp.int32, (N_TOK, N_TOK), 0)
        col_i = lax.broadcasted_iota(jnp.int32, (N_TOK, N_TOK), 1)
        tril = (col_i < row_i).astype(jnp.bfloat16)
        rank = jnp.dot(
            tril, m.astype(jnp.bfloat16), preferred_element_type=jnp.float32
        )
        gates_ref[:, :] = jnp.where(m & (rank < CAPACITY), 1.0, 0.0).astype(
            jnp.bfloat16
        )

        w0 = w_ref[0].astype(jnp.bfloat16)
        w1 = w_ref[1].astype(jnp.bfloat16)

        def chunk_contrib(t):
            rows = pl.ds(t * CHUNK, CHUNK)
            xc = x_ref[rows, :].astype(jnp.bfloat16)
            g = gates_ref[rows, :]
            x0 = xc * g[:, 0:1]
            x1 = xc * g[:, 1:2]
            c = jnp.dot(x0, w0, preferred_element_type=jnp.float32) + jnp.dot(
                x1, w1, preferred_element_type=jnp.float32
            )
            return c.astype(jnp.bfloat16)

        rdmas = []
        for d in range(1, N_DEV):
            t = (p + d) % N_DEV
            rows = pl.ds(t * CHUNK, CHUNK)
            partial_ref[rows, :] = chunk_contrib(t)
            pl.semaphore_wait(credit_sems.at[d - 1], 1)
            rdma = pltpu.make_async_remote_copy(
                src_ref=partial_ref.at[rows, :],
                dst_ref=recv_buf.at[d - 1],
                send_sem=send_sems.at[d - 1],
                recv_sem=recv_sems.at[d - 1],
                device_id=(t,),
                device_id_type=pl.DeviceIdType.MESH,
            )
            rdma.start()
            rdmas.append(rdma)

        acc = chunk_contrib(p).astype(jnp.float32)
        for d in range(1, N_DEV):
            rdmas[d - 1].wait_recv()
            acc = acc + recv_buf[d - 1].astype(jnp.float32)
        out_ref[:, :] = acc

        for rdma in rdmas:
            rdma.wait_send()

    return pl.pallas_call(
        body,
        out_shape=jax.ShapeDtypeStruct((CHUNK, D_OUT), jnp.float32),
        in_specs=[
            pl.BlockSpec(memory_space=pltpu.VMEM),
            pl.BlockSpec(memory_space=pltpu.VMEM),
            pl.BlockSpec(memory_space=pltpu.VMEM),
        ],
        out_specs=pl.BlockSpec(memory_space=pltpu.VMEM),
        scratch_shapes=[
            pltpu.VMEM((N_TOK, D_OUT), jnp.bfloat16),
            pltpu.VMEM((N_TOK, EXPERTS_PER_DEV), jnp.bfloat16),
            pltpu.VMEM((N_DEV - 1, CHUNK, D_OUT), jnp.bfloat16),
            pltpu.SemaphoreType.DMA((N_DEV - 1,)),
            pltpu.SemaphoreType.DMA((N_DEV - 1,)),
            pltpu.SemaphoreType.REGULAR((N_DEV - 1,)),
        ],
        compiler_params=pltpu.CompilerParams(collective_id=0),
    )(x, route_idx, expert_W)
